# baseline (speedup 1.0000x reference)
import dataclasses
import functools

import jax
import jax.numpy as jnp
from jax import lax
from jax.experimental import pallas as pl
from jax.experimental.pallas import tpu as pltpu
from jax.experimental.pallas import tpu_sc as plsc

N = 10000
E = 320000
D = 128

NUM_CORES = 2
NUM_SUBCORES = 16
NUM_WORKERS = NUM_CORES * NUM_SUBCORES
EDGES_PER_WORKER = E // NUM_WORKERS
CHUNK = 40
NUM_CHUNKS = EDGES_PER_WORKER // CHUNK
TOTAL_CHUNKS = E // CHUNK

import numpy as _np
_PV = _np.empty((D,), dtype=_np.int32)
for _k in range(D // 32):
    for _i in range(16):
        _PV[32 * _k + 2 * _i] = 32 * _k + _i
        _PV[32 * _k + 2 * _i + 1] = 32 * _k + 16 + _i
DUMP_SUBCORES = 10
DUMP_ROWS = N // DUMP_SUBCORES

EDGE_BLK = 3200
NUM_EDGE_BLKS = E // EDGE_BLK

_DN = (((1,), (1,)), ((), ()))


def _node_transform_body(h_ref, uw, ub, vw, vb, bw, bb, cw, cb,
                         uh_ref, vh_ref, bh_ref, ch_ref):
    h = h_ref[...]
    uh_ref[...] = lax.dot_general(h, uw[...], _DN,
                                  preferred_element_type=jnp.float32) + ub[...]
    vh_ref[...] = (lax.dot_general(h, vw[...], _DN,
                                   preferred_element_type=jnp.float32)
                   + vb[...]).astype(jnp.bfloat16)
    bh_ref[...] = lax.dot_general(h, bw[...], _DN,
                                  preferred_element_type=jnp.float32) + bb[...]
    ch_ref[...] = (lax.dot_general(h, cw[...], _DN,
                                   preferred_element_type=jnp.float32)
                   + cb[...]).astype(jnp.bfloat16)


def _node_transform(h_in, U_w, U_b, V_w, V_b, B_w, B_b, C_w, C_b):
    out = jax.ShapeDtypeStruct((N, D), jnp.float32)
    out16 = jax.ShapeDtypeStruct((N, D), jnp.bfloat16)
    return pl.pallas_call(
        _node_transform_body,
        out_shape=(out, out16, out, out16),
    )(h_in, U_w, U_b, V_w, V_b, B_w, B_b, C_w, C_b)


def _sc_edge_body(e_hbm, src_hbm, dst_hbm, vc_hbm, bh_hbm,
                  g_hbm, agg_hbm,
                  idx_s0, idx_d0, e0, vc0, bh0, go0,
                  idx_s1, idx_d1, e1, vc1, bh1, go1, m0,
                  sem_in0, sem_in1, sem_g0, sem_g1, sem_w0, sem_w1,
                  agg_sh):
    c = lax.axis_index("c")
    s = lax.axis_index("s")
    wid = c * NUM_SUBCORES + s
    base0 = wid * EDGES_PER_WORKER
    cbase0 = wid * NUM_CHUNKS

    sets = ((idx_s0, idx_d0, e0, vc0, bh0, go0, m0, sem_in0, sem_g0, sem_w0),
            (idx_s1, idx_d1, e1, vc1, bh1, go1, m0, sem_in1, sem_g1, sem_w1))

    @pl.loop(0, CHUNK)
    def _(r):
        for k in range(D // 16):
            m0[r, pl.ds(k * 16, 16)] = jnp.zeros((16,), jnp.float32)

    @pl.when(s < DUMP_SUBCORES)
    def _():
        for j in range(DUMP_ROWS // CHUNK):
            pltpu.sync_copy(
                m0, agg_sh.at[pl.ds(s * DUMP_ROWS + j * CHUNK, CHUNK)])

    plsc.subcore_barrier()

    def issue_in(t, S):
        (idx_s, idx_d, e_b, _, _, _, _, sem_in, _, _) = S
        base = base0 + t * CHUNK
        pltpu.async_copy(src_hbm.at[pl.ds(base, CHUNK)], idx_s, sem_in)
        pltpu.async_copy(dst_hbm.at[pl.ds(base, CHUNK)], idx_d, sem_in)
        pltpu.async_copy(e_hbm.at[pl.ds(base, CHUNK)], e_b, sem_in)

    def wait_in(t, S):
        (idx_s, idx_d, e_b, _, _, _, _, sem_in, _, _) = S
        base = base0 + t * CHUNK
        pltpu.make_async_copy(src_hbm.at[pl.ds(base, CHUNK)], idx_s,
                              sem_in).wait()
        pltpu.make_async_copy(dst_hbm.at[pl.ds(base, CHUNK)], idx_d,
                              sem_in).wait()
        pltpu.make_async_copy(e_hbm.at[pl.ds(base, CHUNK)], e_b,
                              sem_in).wait()

    def issue_gathers(S):
        (idx_s, idx_d, _, vc_b, bh_b, _, _, _, sem_g, _) = S
        pltpu.async_copy(vc_hbm.at[idx_d], vc_b, sem_g)
        pltpu.async_copy(bh_hbm.at[idx_s], bh_b, sem_g)

    def wait_gw(t, S):
        (_, _, _, _, _, go_b, _, _, _, sem_w) = S
        base = base0 + t * CHUNK
        pltpu.make_async_copy(go_b, g_hbm.at[pl.ds(base, CHUNK)],
                              sem_w).wait()

    def finish_chunk(t, S):
        (idx_s, idx_d, e_b, vc_b, bh_b, go_b, m_b, _, sem_g, sem_w) = S
        pltpu.make_async_copy(vc_hbm.at[idx_d], vc_b, sem_g).wait()
        pltpu.make_async_copy(bh_hbm.at[idx_s], bh_b, sem_g).wait()

        @pl.loop(0, CHUNK)
        def _(r):
            for k in range(D // 32):
                vv = plsc.bitcast(vc_b[r, pl.ds(k * 16, 16)], jnp.bfloat16)
                va, vb = plsc.unpack(vv, format=plsc.PackFormat.INTERLEAVED)
                cv = plsc.bitcast(vc_b[r, pl.ds(64 + k * 16, 16)],
                                  jnp.bfloat16)
                ca, cb = plsc.unpack(cv, format=plsc.PackFormat.INTERLEAVED)
                s1 = pl.ds(32 * k, 16)
                s2 = pl.ds(32 * k + 16, 16)
                m_b[r, s1] = va * (1.0 / (1.0 + jnp.exp(-e_b[r, s1])))
                m_b[r, s2] = vb * (1.0 / (1.0 + jnp.exp(-e_b[r, s2])))
                go_b[r, s1] = ca + bh_b[r, s1]
                go_b[r, s2] = cb + bh_b[r, s2]

        base = base0 + t * CHUNK
        pltpu.async_copy(go_b, g_hbm.at[pl.ds(base, CHUNK)], sem_w)
        pltpu.sync_copy(m_b, agg_sh.at[idx_s], add=True)

    issue_in(0, sets[0])
    issue_in(1, sets[1])
    wait_in(0, sets[0])
    issue_gathers(sets[0])

    @pl.loop(0, NUM_CHUNKS, step=2)
    def _(t):
        for p in range(2):
            tt = t + p
            S = sets[p]
            So = sets[1 - p]

            @pl.when(tt + 1 < NUM_CHUNKS)
            def _():
                @pl.when(tt > 0)
                def _():
                    wait_gw(tt - 1, So)
                wait_in(tt + 1, So)
                issue_gathers(So)

            finish_chunk(tt, S)

            @pl.when(tt + 2 < NUM_CHUNKS)
            def _():
                issue_in(tt + 2, S)

    wait_gw(NUM_CHUNKS - 2, sets[0])
    wait_gw(NUM_CHUNKS - 1, sets[1])

    plsc.subcore_barrier()

    @pl.when(s < DUMP_SUBCORES)
    def _():
        pltpu.sync_copy(
            agg_sh.at[pl.ds(s * DUMP_ROWS, DUMP_ROWS)],
            agg_hbm.at[c, pl.ds(s * DUMP_ROWS, DUMP_ROWS)])


def _sc_edge_pass(e_in, src, dst, vc, bh):
    mesh = plsc.VectorSubcoreMesh(core_axis_name="c", subcore_axis_name="s",
                                  num_cores=NUM_CORES,
                                  num_subcores=NUM_SUBCORES)
    buf_set = [
        pltpu.VMEM((CHUNK,), jnp.int32),
        pltpu.VMEM((CHUNK,), jnp.int32),
        pltpu.VMEM((CHUNK, D), jnp.float32),
        pltpu.VMEM((CHUNK, D), jnp.int32),
        pltpu.VMEM((CHUNK, D), jnp.float32),
        pltpu.VMEM((CHUNK, D), jnp.float32),
    ]
    cp = pltpu.CompilerParams()
    if "needs_layout_passes" in pltpu.CompilerParams.__dataclass_fields__:
        cp = dataclasses.replace(cp, needs_layout_passes=False)
    kernel = pl.kernel(
        _sc_edge_body,
        out_type=(jax.ShapeDtypeStruct((E, D), jnp.float32),
                  jax.ShapeDtypeStruct((NUM_CORES, N, D), jnp.float32)),
        mesh=mesh,
        compiler_params=cp,
        scratch_types=buf_set + buf_set + [
            pltpu.VMEM((CHUNK, D), jnp.float32),
            pltpu.SemaphoreType.DMA,
            pltpu.SemaphoreType.DMA,
            pltpu.SemaphoreType.DMA,
            pltpu.SemaphoreType.DMA,
            pltpu.SemaphoreType.DMA,
            pltpu.SemaphoreType.DMA,
            pltpu.VMEM_SHARED((N, D), jnp.float32),
        ],
    )
    return kernel(e_in, src, dst, vc, bh)


def _e_stats_body(e_ref, g_ref, aw, ab, out_ref):
    i = pl.program_id(0)
    ae = lax.dot_general(e_ref[...].astype(jnp.bfloat16),
                         aw[...].astype(jnp.bfloat16), _DN,
                         preferred_element_type=jnp.float32) + ab[...]
    pre = ae + g_ref[...].astype(jnp.float32)
    ssum = jnp.sum(pre, axis=0)
    ssq = jnp.sum(pre * pre, axis=0)

    @pl.when(i == 0)
    def _():
        out_ref[...] = jnp.zeros_like(out_ref)

    out_ref[0, :] += ssum
    out_ref[1, :] += ssq


def _e_stats(e_in, g, A_w, A_b):
    return pl.pallas_call(
        _e_stats_body,
        grid=(NUM_EDGE_BLKS,),
        in_specs=[
            pl.BlockSpec((EDGE_BLK, D), lambda i: (i, 0)),
            pl.BlockSpec((EDGE_BLK, D), lambda i: (i, 0)),
            pl.BlockSpec((D, D), lambda i: (0, 0)),
            pl.BlockSpec((D,), lambda i: (0,)),
        ],
        out_specs=pl.BlockSpec((8, D), lambda i: (0, 0)),
        out_shape=jax.ShapeDtypeStruct((8, D), jnp.float32),
    )(e_in, g, A_w, A_b)


def _e_final_body(e_ref, g_ref, aw, ab, stats_ref, gamma_ref, beta_ref,
                  out_ref):
    ae = lax.dot_general(e_ref[...].astype(jnp.bfloat16),
                         aw[...].astype(jnp.bfloat16), _DN,
                         preferred_element_type=jnp.float32) + ab[...]
    pre = ae + g_ref[...].astype(jnp.float32)
    mu = stats_ref[0, :] * (1.0 / E)
    var = stats_ref[1, :] * (1.0 / E) - mu * mu
    inv = gamma_ref[...] * lax.rsqrt(var + 1e-5)
    bn = (pre - mu) * inv + beta_ref[...]
    out_ref[...] = e_ref[...] + jnp.maximum(bn, 0.0)


def _e_final(e_in, g, A_w, A_b, stats, e_gamma, e_beta):
    return pl.pallas_call(
        _e_final_body,
        grid=(NUM_EDGE_BLKS,),
        in_specs=[
            pl.BlockSpec((EDGE_BLK, D), lambda i: (i, 0)),
            pl.BlockSpec((EDGE_BLK, D), lambda i: (i, 0)),
            pl.BlockSpec((D, D), lambda i: (0, 0)),
            pl.BlockSpec((D,), lambda i: (0,)),
            pl.BlockSpec((8, D), lambda i: (0, 0)),
            pl.BlockSpec((D,), lambda i: (0,)),
            pl.BlockSpec((D,), lambda i: (0,)),
        ],
        out_specs=pl.BlockSpec((EDGE_BLK, D), lambda i: (i, 0)),
        out_shape=jax.ShapeDtypeStruct((E, D), jnp.float32),
    )(e_in, g, A_w, A_b, stats, e_gamma, e_beta)


def _h_final_body(h_ref, uh_ref, a0_ref, a1_ref, gamma_ref, beta_ref,
                  out_ref):
    t = uh_ref[...] + a0_ref[...] + a1_ref[...]
    mu = jnp.mean(t, axis=0)
    var = jnp.mean(t * t, axis=0) - mu * mu
    inv = gamma_ref[...] * lax.rsqrt(var + 1e-5)
    bn = (t - mu) * inv + beta_ref[...]
    out_ref[...] = h_ref[...] + jnp.maximum(bn, 0.0)


def _h_final(h_in, uh, agg, h_gamma, h_beta):
    return pl.pallas_call(
        _h_final_body,
        out_shape=jax.ShapeDtypeStruct((N, D), jnp.float32),
    )(h_in, uh, agg[0], agg[1], h_gamma, h_beta)


@jax.jit
def kernel(h_in, e_in, edge_index, U_w, U_b, V_w, V_b, A_w, A_b, B_w, B_b,
           C_w, C_b, h_gamma, h_beta, e_gamma, e_beta):
    src = edge_index[0]
    dst = edge_index[1]
    pv = jnp.asarray(_PV)
    uh, vh, bh, ch = _node_transform(h_in, U_w, U_b,
                                     V_w[pv], V_b[pv],
                                     B_w, B_b, C_w[pv], C_b[pv])
    vc = jnp.concatenate(
        [lax.bitcast_convert_type(vh.reshape(N, D // 2, 2), jnp.int32),
         lax.bitcast_convert_type(ch.reshape(N, D // 2, 2), jnp.int32)],
        axis=1)
    g, agg = _sc_edge_pass(e_in, src, dst, vc, bh)
    stats = _e_stats(e_in, g, A_w, A_b)
    e_out = _e_final(e_in, g, A_w, A_b, stats, e_gamma, e_beta)
    h_out = _h_final(h_in, uh, agg, h_gamma, h_beta)
    return (h_out, e_out)


# speedup vs baseline: 1.0738x; 1.0738x over previous
import dataclasses
import functools

import jax
import jax.numpy as jnp
from jax import lax
from jax.experimental import pallas as pl
from jax.experimental.pallas import tpu as pltpu
from jax.experimental.pallas import tpu_sc as plsc

N = 10000
E = 320000
D = 128

NUM_CORES = 2
NUM_SUBCORES = 16
NUM_WORKERS = NUM_CORES * NUM_SUBCORES
EDGES_PER_WORKER = E // NUM_WORKERS
CHUNK = 40
NUM_CHUNKS = EDGES_PER_WORKER // CHUNK
TOTAL_CHUNKS = E // CHUNK

import numpy as _np
_PV = _np.empty((D,), dtype=_np.int32)
for _k in range(D // 32):
    for _i in range(16):
        _PV[32 * _k + 2 * _i] = 32 * _k + _i
        _PV[32 * _k + 2 * _i + 1] = 32 * _k + 16 + _i
DUMP_SUBCORES = 10
DUMP_ROWS = N // DUMP_SUBCORES

EDGE_BLK = 3200
NUM_EDGE_BLKS = E // EDGE_BLK

_DN = (((1,), (1,)), ((), ()))


def _node_transform_body(h_ref, uw, ub, vw, vb, bw, bb, cw, cb,
                         uh_ref, vh_ref, bh_ref, ch_ref):
    h = h_ref[...]
    uh_ref[...] = lax.dot_general(h, uw[...], _DN,
                                  preferred_element_type=jnp.float32) + ub[...]
    vh_ref[...] = (lax.dot_general(h, vw[...], _DN,
                                   preferred_element_type=jnp.float32)
                   + vb[...]).astype(jnp.bfloat16)
    bh_ref[...] = lax.dot_general(h, bw[...], _DN,
                                  preferred_element_type=jnp.float32) + bb[...]
    ch_ref[...] = (lax.dot_general(h, cw[...], _DN,
                                   preferred_element_type=jnp.float32)
                   + cb[...]).astype(jnp.bfloat16)


def _node_transform(h_in, U_w, U_b, V_w, V_b, B_w, B_b, C_w, C_b):
    out = jax.ShapeDtypeStruct((N, D), jnp.float32)
    out16 = jax.ShapeDtypeStruct((N, D), jnp.bfloat16)
    return pl.pallas_call(
        _node_transform_body,
        out_shape=(out, out16, out, out16),
    )(h_in, U_w, U_b, V_w, V_b, B_w, B_b, C_w, C_b)


def _sc_edge_body(e_hbm, src_hbm, dst_hbm, vc_hbm, bh_hbm,
                  g_hbm, agg_hbm,
                  idx_s0, idx_d0, e0, vc0, bh0, go0,
                  idx_s1, idx_d1, e1, vc1, bh1, go1, m0,
                  idx_c0, idx_c1,
                  sem_in0, sem_in1, sem_g0, sem_g1, sem_w0, sem_w1, sem_sc,
                  sem_ic0, sem_ic1,
                  agg_sh):
    c = lax.axis_index("c")
    s = lax.axis_index("s")
    wid = c * NUM_SUBCORES + s
    base0 = wid * EDGES_PER_WORKER
    cbase0 = wid * NUM_CHUNKS

    sets = ((idx_s0, idx_d0, e0, vc0, bh0, go0, m0, sem_in0, sem_g0, sem_w0,
             idx_c0, sem_ic0),
            (idx_s1, idx_d1, e1, vc1, bh1, go1, m0, sem_in1, sem_g1, sem_w1,
             idx_c1, sem_ic1))

    def wait_scatter():
        pltpu.make_async_copy(m0, agg_sh.at[idx_c0], sem_sc).wait()

    @pl.loop(0, CHUNK)
    def _(r):
        for k in range(D // 16):
            m0[r, pl.ds(k * 16, 16)] = jnp.zeros((16,), jnp.float32)

    @pl.when(s < DUMP_SUBCORES)
    def _():
        for j in range(DUMP_ROWS // CHUNK):
            pltpu.sync_copy(
                m0, agg_sh.at[pl.ds(s * DUMP_ROWS + j * CHUNK, CHUNK)])

    plsc.subcore_barrier()

    def issue_in(t, S):
        (idx_s, idx_d, e_b, _, _, _, _, sem_in, _, _, _, _) = S
        base = base0 + t * CHUNK
        pltpu.async_copy(src_hbm.at[pl.ds(base, CHUNK)], idx_s, sem_in)
        pltpu.async_copy(dst_hbm.at[pl.ds(base, CHUNK)], idx_d, sem_in)
        pltpu.async_copy(e_hbm.at[pl.ds(base, CHUNK)], e_b, sem_in)

    def wait_in(t, S):
        (idx_s, idx_d, e_b, _, _, _, _, sem_in, _, _, _, _) = S
        base = base0 + t * CHUNK
        pltpu.make_async_copy(src_hbm.at[pl.ds(base, CHUNK)], idx_s,
                              sem_in).wait()
        pltpu.make_async_copy(dst_hbm.at[pl.ds(base, CHUNK)], idx_d,
                              sem_in).wait()
        pltpu.make_async_copy(e_hbm.at[pl.ds(base, CHUNK)], e_b,
                              sem_in).wait()

    def issue_gathers(S):
        (idx_s, idx_d, _, vc_b, bh_b, _, _, _, sem_g, _, _, _) = S
        pltpu.async_copy(vc_hbm.at[idx_d], vc_b, sem_g)
        pltpu.async_copy(bh_hbm.at[idx_s], bh_b, sem_g)

    def wait_gw(t, S):
        (_, _, _, _, _, go_b, _, _, _, sem_w, _, _) = S
        base = base0 + t * CHUNK
        pltpu.make_async_copy(go_b, g_hbm.at[pl.ds(base, CHUNK)],
                              sem_w).wait()

    def finish_chunk(t, S):
        (idx_s, idx_d, e_b, vc_b, bh_b, go_b, m_b, _, sem_g, sem_w,
         idx_c, sem_ic) = S
        base = base0 + t * CHUNK
        pltpu.async_copy(src_hbm.at[pl.ds(base, CHUNK)], idx_c, sem_ic)
        pltpu.make_async_copy(vc_hbm.at[idx_d], vc_b, sem_g).wait()
        pltpu.make_async_copy(bh_hbm.at[idx_s], bh_b, sem_g).wait()

        @pl.when(t >= 1)
        def _():
            wait_scatter()

        @pl.loop(0, CHUNK)
        def _(r):
            for k in range(D // 32):
                vv = plsc.bitcast(vc_b[r, pl.ds(k * 16, 16)], jnp.bfloat16)
                va, vb = plsc.unpack(vv, format=plsc.PackFormat.INTERLEAVED)
                cv = plsc.bitcast(vc_b[r, pl.ds(64 + k * 16, 16)],
                                  jnp.bfloat16)
                ca, cb = plsc.unpack(cv, format=plsc.PackFormat.INTERLEAVED)
                s1 = pl.ds(32 * k, 16)
                s2 = pl.ds(32 * k + 16, 16)
                m_b[r, s1] = va * (1.0 / (1.0 + jnp.exp(-e_b[r, s1])))
                m_b[r, s2] = vb * (1.0 / (1.0 + jnp.exp(-e_b[r, s2])))
                go_b[r, s1] = ca + bh_b[r, s1]
                go_b[r, s2] = cb + bh_b[r, s2]

        pltpu.async_copy(go_b, g_hbm.at[pl.ds(base, CHUNK)], sem_w)
        pltpu.make_async_copy(src_hbm.at[pl.ds(base, CHUNK)], idx_c,
                              sem_ic).wait()
        pltpu.async_copy(m_b, agg_sh.at[idx_c], sem_sc, add=True)

    issue_in(0, sets[0])
    issue_in(1, sets[1])
    wait_in(0, sets[0])
    issue_gathers(sets[0])

    @pl.loop(0, NUM_CHUNKS, step=2)
    def _(t):
        for p in range(2):
            tt = t + p
            S = sets[p]
            So = sets[1 - p]

            @pl.when(tt + 1 < NUM_CHUNKS)
            def _():
                @pl.when(tt > 0)
                def _():
                    wait_gw(tt - 1, So)
                wait_in(tt + 1, So)
                issue_gathers(So)

            finish_chunk(tt, S)

            @pl.when(tt + 2 < NUM_CHUNKS)
            def _():
                issue_in(tt + 2, S)

    wait_gw(NUM_CHUNKS - 2, sets[0])
    wait_gw(NUM_CHUNKS - 1, sets[1])
    wait_scatter()

    plsc.subcore_barrier()

    @pl.when(s < DUMP_SUBCORES)
    def _():
        pltpu.sync_copy(
            agg_sh.at[pl.ds(s * DUMP_ROWS, DUMP_ROWS)],
            agg_hbm.at[c, pl.ds(s * DUMP_ROWS, DUMP_ROWS)])


def _sc_edge_pass(e_in, src, dst, vc, bh):
    mesh = plsc.VectorSubcoreMesh(core_axis_name="c", subcore_axis_name="s",
                                  num_cores=NUM_CORES,
                                  num_subcores=NUM_SUBCORES)
    buf_set = [
        pltpu.VMEM((CHUNK,), jnp.int32),
        pltpu.VMEM((CHUNK,), jnp.int32),
        pltpu.VMEM((CHUNK, D), jnp.float32),
        pltpu.VMEM((CHUNK, D), jnp.int32),
        pltpu.VMEM((CHUNK, D), jnp.float32),
        pltpu.VMEM((CHUNK, D), jnp.float32),
    ]
    cp = pltpu.CompilerParams()
    if "needs_layout_passes" in pltpu.CompilerParams.__dataclass_fields__:
        cp = dataclasses.replace(cp, needs_layout_passes=False)
    kernel = pl.kernel(
        _sc_edge_body,
        out_type=(jax.ShapeDtypeStruct((E, D), jnp.float32),
                  jax.ShapeDtypeStruct((NUM_CORES, N, D), jnp.float32)),
        mesh=mesh,
        compiler_params=cp,
        scratch_types=buf_set + buf_set + [
            pltpu.VMEM((CHUNK, D), jnp.float32),
            pltpu.VMEM((CHUNK,), jnp.int32),
            pltpu.VMEM((CHUNK,), jnp.int32),
            pltpu.SemaphoreType.DMA,
            pltpu.SemaphoreType.DMA,
            pltpu.SemaphoreType.DMA,
            pltpu.SemaphoreType.DMA,
            pltpu.SemaphoreType.DMA,
            pltpu.SemaphoreType.DMA,
            pltpu.SemaphoreType.DMA,
            pltpu.SemaphoreType.DMA,
            pltpu.SemaphoreType.DMA,
            pltpu.VMEM_SHARED((N, D), jnp.float32),
        ],
    )
    return kernel(e_in, src, dst, vc, bh)


def _e_stats_body(e_ref, g_ref, aw, ab, out_ref):
    i = pl.program_id(0)
    ae = lax.dot_general(e_ref[...], aw[...], _DN,
                         preferred_element_type=jnp.float32) + ab[...]
    pre = ae + g_ref[...].astype(jnp.float32)
    ssum = jnp.sum(pre, axis=0)
    ssq = jnp.sum(pre * pre, axis=0)

    @pl.when(i == 0)
    def _():
        out_ref[...] = jnp.zeros_like(out_ref)

    out_ref[0, :] += ssum
    out_ref[1, :] += ssq


def _e_stats(e_in, g, A_w, A_b):
    return pl.pallas_call(
        _e_stats_body,
        grid=(NUM_EDGE_BLKS,),
        in_specs=[
            pl.BlockSpec((EDGE_BLK, D), lambda i: (i, 0)),
            pl.BlockSpec((EDGE_BLK, D), lambda i: (i, 0)),
            pl.BlockSpec((D, D), lambda i: (0, 0)),
            pl.BlockSpec((D,), lambda i: (0,)),
        ],
        out_specs=pl.BlockSpec((8, D), lambda i: (0, 0)),
        out_shape=jax.ShapeDtypeStruct((8, D), jnp.float32),
    )(e_in, g, A_w, A_b)


def _e_final_body(e_ref, g_ref, aw, ab, stats_ref, gamma_ref, beta_ref,
                  out_ref):
    ae = lax.dot_general(e_ref[...], aw[...], _DN,
                         preferred_element_type=jnp.float32) + ab[...]
    pre = ae + g_ref[...].astype(jnp.float32)
    mu = stats_ref[0, :] * (1.0 / E)
    var = stats_ref[1, :] * (1.0 / E) - mu * mu
    inv = gamma_ref[...] * lax.rsqrt(var + 1e-5)
    bn = (pre - mu) * inv + beta_ref[...]
    out_ref[...] = e_ref[...] + jnp.maximum(bn, 0.0)


def _e_final(e_in, g, A_w, A_b, stats, e_gamma, e_beta):
    return pl.pallas_call(
        _e_final_body,
        grid=(NUM_EDGE_BLKS,),
        in_specs=[
            pl.BlockSpec((EDGE_BLK, D), lambda i: (i, 0)),
            pl.BlockSpec((EDGE_BLK, D), lambda i: (i, 0)),
            pl.BlockSpec((D, D), lambda i: (0, 0)),
            pl.BlockSpec((D,), lambda i: (0,)),
            pl.BlockSpec((8, D), lambda i: (0, 0)),
            pl.BlockSpec((D,), lambda i: (0,)),
            pl.BlockSpec((D,), lambda i: (0,)),
        ],
        out_specs=pl.BlockSpec((EDGE_BLK, D), lambda i: (i, 0)),
        out_shape=jax.ShapeDtypeStruct((E, D), jnp.float32),
    )(e_in, g, A_w, A_b, stats, e_gamma, e_beta)


def _h_final_body(h_ref, uh_ref, a0_ref, a1_ref, gamma_ref, beta_ref,
                  out_ref):
    t = uh_ref[...] + a0_ref[...] + a1_ref[...]
    mu = jnp.mean(t, axis=0)
    var = jnp.mean(t * t, axis=0) - mu * mu
    inv = gamma_ref[...] * lax.rsqrt(var + 1e-5)
    bn = (t - mu) * inv + beta_ref[...]
    out_ref[...] = h_ref[...] + jnp.maximum(bn, 0.0)


def _h_final(h_in, uh, agg, h_gamma, h_beta):
    return pl.pallas_call(
        _h_final_body,
        out_shape=jax.ShapeDtypeStruct((N, D), jnp.float32),
    )(h_in, uh, agg[0], agg[1], h_gamma, h_beta)


@jax.jit
def kernel(h_in, e_in, edge_index, U_w, U_b, V_w, V_b, A_w, A_b, B_w, B_b,
           C_w, C_b, h_gamma, h_beta, e_gamma, e_beta):
    src = edge_index[0]
    dst = edge_index[1]
    pv = jnp.asarray(_PV)
    uh, vh, bh, ch = _node_transform(h_in, U_w, U_b,
                                     V_w[pv], V_b[pv],
                                     B_w, B_b, C_w[pv], C_b[pv])
    vc = jnp.concatenate(
        [lax.bitcast_convert_type(vh.reshape(N, D // 2, 2), jnp.int32),
         lax.bitcast_convert_type(ch.reshape(N, D // 2, 2), jnp.int32)],
        axis=1)
    g, agg = _sc_edge_pass(e_in, src, dst, vc, bh)
    stats = _e_stats(e_in, g, A_w, A_b)
    e_out = _e_final(e_in, g, A_w, A_b, stats, e_gamma, e_beta)
    h_out = _h_final(h_in, uh, agg, h_gamma, h_beta)
    return (h_out, e_out)
